# Initial kernel scaffold; baseline (speedup 1.0000x reference)
import jax
import jax.numpy as jnp
from jax.experimental import pallas as pl


def kernel(sppm_pool_w, sppm_pool_b, sppm_out_w, sppm_out_b, out_w_blocks, out_b_blocks, out_w_proj, out_b_proj, level0_w, level0_b, level0_w_att, level0_b_att, level1_w, level1_b, level1_w_att, level1_b_att, input_0, input_1, input_2, input_3, input_4, input_5):
    raise NotImplementedError("write your pallas kernel here")


# speedup vs baseline: 1.1493x; 1.1493x over previous
import functools

import jax
import jax.numpy as jnp
import numpy as np
from jax.experimental import pallas as pl
from jax.experimental.pallas import tpu as pltpu

_VMEM_LIMIT = 64 * 1024 * 1024


def _ru(x, m):
    return ((x + m - 1) // m) * m


def _bilinear_matrix(out, inn):
    c = (np.arange(out, dtype=np.float64) + 0.5) * (inn / out) - 0.5
    c = np.clip(c, 0.0, inn - 1)
    lo = np.floor(c).astype(np.int32)
    hi = np.minimum(lo + 1, inn - 1)
    f = (c - lo).astype(np.float32)
    R = np.zeros((out, inn), np.float32)
    R[np.arange(out), lo] += 1.0 - f
    R[np.arange(out), hi] += f
    return R


def _pad_flat(x, p_ext, cpad=None):
    B, H, W, C = x.shape
    if cpad is not None and cpad > C:
        x = jnp.pad(x, ((0, 0), (0, 0), (0, 0), (0, cpad - C)))
    xp = jnp.pad(x.astype(jnp.bfloat16), ((0, 0), (1, 1), (1, 1), (0, 0)))
    flat = xp.reshape(B, (H + 2) * (W + 2), xp.shape[-1])
    return jnp.pad(flat, ((0, 0), (0, p_ext - flat.shape[1]), (0, 0)))


def _offs(W2):
    return [dy * W2 + dx for dy in range(3) for dx in range(3)]


def _sppm_body(a_ref, pw_ref, pb_ref, u_ref, ow_ref, ob_ref, o_ref,
               *, cin, nb, sizes):
    pb = pb_ref[...]
    ys = []
    r0 = 0
    for bi, ps in enumerate(sizes):
        n = ps * ps
        w = pw_ref[bi * cin:(bi + 1) * cin, :]
        bias = pw_ref[nb * cin + bi:nb * cin + bi + 1, :].astype(jnp.float32)
        y = jnp.dot(a_ref[r0:r0 + n, :], w, preferred_element_type=jnp.float32)
        ys.append(jnp.maximum(y + bias + pb, 0.0))
        r0 += n
    ycat = jnp.concatenate(ys, axis=0)
    fused = jnp.dot(u_ref[...], ycat, preferred_element_type=jnp.float32,
                    precision=jax.lax.Precision.HIGHEST)
    out = jnp.dot(fused.astype(jnp.bfloat16), ow_ref[...],
                  preferred_element_type=jnp.float32) + ob_ref[...]
    o_ref[...] = jnp.maximum(out, 0.0).astype(o_ref.dtype)


def _sppm(x5, pool_w, pool_b, out_w, out_b, sizes):
    B, Ht, Wt, cin = x5.shape
    C = out_b.shape[-1]
    nb = len(sizes)
    npx = sum(ps * ps for ps in sizes)
    pix = [x5[:, ::Ht // ps, ::Wt // ps].reshape(B, ps * ps, cin)
           for ps in sizes]
    A = jnp.concatenate(pix, axis=1).astype(jnp.bfloat16)
    U = np.concatenate(
        [np.kron(_bilinear_matrix(Ht, ps), _bilinear_matrix(Wt, ps))
         for ps in sizes], axis=1)
    U = jnp.asarray(U, jnp.float32)
    out = pl.pallas_call(
        functools.partial(_sppm_body, cin=cin, nb=nb, sizes=sizes),
        out_shape=jax.ShapeDtypeStruct((B, Ht * Wt, C), jnp.bfloat16),
        grid=(B,),
        in_specs=[
            pl.BlockSpec((None, npx, cin), lambda b: (b, 0, 0)),
            pl.BlockSpec(pool_w.shape, lambda b: (0, 0)),
            pl.BlockSpec((1, C), lambda b: (0, 0)),
            pl.BlockSpec((Ht * Wt, npx), lambda b: (0, 0)),
            pl.BlockSpec((C, C), lambda b: (0, 0)),
            pl.BlockSpec((1, C), lambda b: (0, 0)),
        ],
        out_specs=pl.BlockSpec((None, Ht * Wt, C), lambda b: (b, 0, 0)),
        compiler_params=pltpu.CompilerParams(
            dimension_semantics=("parallel",), vmem_limit_bytes=_VMEM_LIMIT),
    )(A, pool_w, pool_b, U, out_w, out_b)
    return out.reshape(B, Ht, Wt, C)


def _fuse_body(x_ref, w_ref, b_ref, wa_ref, ba_ref, o_ref, *, H, W, C):
    W2 = W + 2
    P = (H + 2) * W2
    P_ext = x_ref.shape[0]
    offs = _offs(W2)

    a = jnp.concatenate([x_ref[off:off + P, :] for off in offs], axis=-1)
    y = jnp.dot(a, w_ref[...], preferred_element_type=jnp.float32) + b_ref[...]
    y = jnp.maximum(y, 0.0)
    x1 = y[:, :C]
    x2 = y[:, C:]

    att = jnp.concatenate(
        [jnp.mean(x1, axis=-1, keepdims=True),
         jnp.max(x1, axis=-1, keepdims=True),
         jnp.mean(x2, axis=-1, keepdims=True),
         jnp.max(x2, axis=-1, keepdims=True)], axis=-1)

    q = jax.lax.broadcasted_iota(jnp.int32, (P, 1), 0)
    oy = q // W2
    ox = q - oy * W2
    mask = jnp.logical_and(oy < H, ox < W)
    lead = W2 + 1
    trail = P_ext - P - lead
    att_full = jnp.concatenate(
        [jnp.zeros((lead, 4), jnp.float32),
         jnp.where(mask, att, 0.0),
         jnp.zeros((trail, 4), jnp.float32)], axis=0)
    acc = jnp.zeros((P, 1), jnp.float32)
    for s, off in enumerate(offs):
        acc = acc + jnp.sum(att_full[off:off + P, :] * wa_ref[s],
                            axis=-1, keepdims=True)
    alpha = jax.nn.sigmoid(acc + ba_ref[...])
    o_ref[...] = (x1 * alpha + x2 * (1.0 - alpha)).astype(o_ref.dtype)


def _fuse_level(feat, x_prev, w, bfull, wa, ba):
    B, H, W, cin = feat.shape
    C = bfull.shape[-1] // 2
    cinp = (w.shape[0] - 9 * C) // 9
    W2 = W + 2
    P = (H + 2) * W2
    p_ext = _ru(P + 2 * W2 + 2, 8)
    if cinp > cin:
        feat = jnp.pad(feat, ((0, 0), (0, 0), (0, 0), (0, cinp - cin)))
    x_up = jnp.repeat(jnp.repeat(x_prev, 2, axis=1), 2, axis=2)
    xcat = jnp.concatenate([feat.astype(jnp.bfloat16),
                            x_up.astype(jnp.bfloat16)], axis=-1)
    xp = _pad_flat(xcat, p_ext)
    K1 = 9 * cinp
    w_cat = jnp.concatenate(
        [w[:K1].reshape(9, cinp, 2 * C), w[K1:].reshape(9, C, 2 * C)],
        axis=1).reshape(9 * (cinp + C), 2 * C)
    out = pl.pallas_call(
        functools.partial(_fuse_body, H=H, W=W, C=C),
        out_shape=jax.ShapeDtypeStruct((B, P, C), jnp.bfloat16),
        grid=(B,),
        in_specs=[
            pl.BlockSpec((None, p_ext, cinp + C), lambda b: (b, 0, 0)),
            pl.BlockSpec((9 * (cinp + C), 2 * C), lambda b: (0, 0)),
            pl.BlockSpec((1, 2 * C), lambda b: (0, 0)),
            pl.BlockSpec((9, 1, 4), lambda b: (0, 0, 0)),
            pl.BlockSpec((1, 1), lambda b: (0, 0)),
        ],
        out_specs=pl.BlockSpec((None, P, C), lambda b: (b, 0, 0)),
        compiler_params=pltpu.CompilerParams(
            dimension_semantics=("parallel",), vmem_limit_bytes=_VMEM_LIMIT),
    )(xp, w_cat, bfull, wa, ba)
    return out.reshape(B, H + 2, W2, C)[:, :H, :W, :]


def _out_body(x_ref, w_ref, b_ref, wp_ref, bp_ref, s_ref, c_ref, *, H, W):
    W2 = W + 2
    P = (H + 2) * W2
    P_ext = x_ref.shape[0]
    C = x_ref.shape[-1]
    L = w_ref.shape[0]
    ncls = wp_ref.shape[-1]
    offs = _offs(W2)
    lead = W2 + 1
    trail = P_ext - P - lead

    q = jax.lax.broadcasted_iota(jnp.int32, (P, 1), 0)
    oy = q // W2
    ox = q - oy * W2
    mask = jnp.logical_and(oy < H, ox < W)

    def conv(src, wl, bl):
        a = jnp.concatenate([src[off:off + P, :] for off in offs], axis=-1)
        return jnp.maximum(
            jnp.dot(a, wl, preferred_element_type=jnp.float32) + bl, 0.0)

    y = conv(x_ref, w_ref[0], b_ref[0])
    for l in range(1, L):
        yb = jnp.where(mask, y, 0.0).astype(jnp.bfloat16)
        y_full = jnp.concatenate(
            [jnp.zeros((lead, C), jnp.bfloat16),
             yb,
             jnp.zeros((trail, C), jnp.bfloat16)], axis=0)
        y = conv(y_full, w_ref[l], b_ref[l])

    logits = jnp.dot(y.astype(jnp.bfloat16), wp_ref[...],
                     preferred_element_type=jnp.float32) + bp_ref[...]
    m = jnp.max(logits, axis=-1, keepdims=True)
    denom = jnp.sum(jnp.exp(logits - m), axis=-1, keepdims=True)
    s_ref[...] = (1.0 / denom).astype(s_ref.dtype)
    cidx = jax.lax.broadcasted_iota(jnp.int32, logits.shape, 1)
    c_ref[...] = jnp.min(jnp.where(logits == m, cidx, ncls),
                         axis=-1, keepdims=True).astype(c_ref.dtype)


def _out_chain(x, w_blocks, b_blocks, w_proj, b_proj):
    B, H, W, C = x.shape
    ncls = w_proj.shape[1]
    L = w_blocks.shape[0]
    W2 = W + 2
    P = (H + 2) * W2
    p_ext = _ru(P + 2 * W2 + 2, 8)
    x_pad = _pad_flat(x, p_ext)
    scores, classes = pl.pallas_call(
        functools.partial(_out_body, H=H, W=W),
        out_shape=(jax.ShapeDtypeStruct((B, P, 1), jnp.float32),
                   jax.ShapeDtypeStruct((B, P, 1), jnp.int32)),
        grid=(B,),
        in_specs=[
            pl.BlockSpec((None, p_ext, C), lambda b: (b, 0, 0)),
            pl.BlockSpec((L, 9 * C, C), lambda b: (0, 0, 0)),
            pl.BlockSpec((L, 1, C), lambda b: (0, 0, 0)),
            pl.BlockSpec((C, ncls), lambda b: (0, 0)),
            pl.BlockSpec((1, ncls), lambda b: (0, 0)),
        ],
        out_specs=(pl.BlockSpec((None, P, 1), lambda b: (b, 0, 0)),
                   pl.BlockSpec((None, P, 1), lambda b: (b, 0, 0))),
        compiler_params=pltpu.CompilerParams(
            dimension_semantics=("parallel",), vmem_limit_bytes=_VMEM_LIMIT),
    )(x_pad, w_blocks, b_blocks, w_proj, b_proj)
    scores = scores.reshape(B, H + 2, W2)[:, :H, :W]
    classes = classes.reshape(B, H + 2, W2)[:, :H, :W]
    return scores, classes


def kernel(sppm_pool_w, sppm_pool_b, sppm_out_w, sppm_out_b,
           out_w_blocks, out_b_blocks, out_w_proj, out_b_proj,
           level0_w, level0_b, level0_w_att, level0_b_att,
           level1_w, level1_b, level1_w_att, level1_b_att,
           input_0, input_1, input_2, input_3, input_4, input_5):
    x5 = jnp.transpose(input_5, (0, 2, 3, 1))
    x4 = jnp.transpose(input_4, (0, 2, 3, 1))
    x3 = jnp.transpose(input_3, (0, 2, 3, 1))

    x = _sppm(x5, sppm_pool_w, sppm_pool_b, sppm_out_w, sppm_out_b, (1, 2, 4))
    x = _fuse_level(x4, x, level0_w, level0_b, level0_w_att, level0_b_att)
    x = _fuse_level(x3, x, level1_w, level1_b, level1_w_att, level1_b_att)
    scores, classes = _out_chain(x, out_w_blocks, out_b_blocks,
                                 out_w_proj, out_b_proj)

    H0, W0 = input_0.shape[2], input_0.shape[3]
    ry, rx = H0 // scores.shape[1], W0 // scores.shape[2]
    scores = jnp.repeat(jnp.repeat(scores, ry, axis=1), rx, axis=2)
    classes = jnp.repeat(jnp.repeat(classes, ry, axis=1), rx, axis=2)
    return scores, classes


# speedup vs baseline: 1.1792x; 1.0260x over previous
import functools

import jax
import jax.numpy as jnp
import numpy as np
from jax.experimental import pallas as pl
from jax.experimental.pallas import tpu as pltpu

_VMEM_LIMIT = 64 * 1024 * 1024


def _ru(x, m):
    return ((x + m - 1) // m) * m


def _bilinear_matrix(out, inn):
    c = (np.arange(out, dtype=np.float64) + 0.5) * (inn / out) - 0.5
    c = np.clip(c, 0.0, inn - 1)
    lo = np.floor(c).astype(np.int32)
    hi = np.minimum(lo + 1, inn - 1)
    f = (c - lo).astype(np.float32)
    R = np.zeros((out, inn), np.float32)
    R[np.arange(out), lo] += 1.0 - f
    R[np.arange(out), hi] += f
    return R


def _pad_flat(x, p_ext, cpad=None):
    B, H, W, C = x.shape
    if cpad is not None and cpad > C:
        x = jnp.pad(x, ((0, 0), (0, 0), (0, 0), (0, cpad - C)))
    xp = jnp.pad(x.astype(jnp.bfloat16), ((0, 0), (1, 1), (1, 1), (0, 0)))
    flat = xp.reshape(B, (H + 2) * (W + 2), xp.shape[-1])
    return jnp.pad(flat, ((0, 0), (0, p_ext - flat.shape[1]), (0, 0)))


def _offs(W2):
    return [dy * W2 + dx for dy in range(3) for dx in range(3)]


def _sppm_body(a_ref, pw_ref, pb_ref, u_ref, ow_ref, ob_ref, o_ref,
               *, cin, nb, sizes):
    pb = pb_ref[...]
    ys = []
    r0 = 0
    for bi, ps in enumerate(sizes):
        n = ps * ps
        w = pw_ref[bi * cin:(bi + 1) * cin, :]
        bias = pw_ref[nb * cin + bi:nb * cin + bi + 1, :].astype(jnp.float32)
        y = jnp.dot(a_ref[r0:r0 + n, :], w, preferred_element_type=jnp.float32)
        ys.append(jnp.maximum(y + bias + pb, 0.0))
        r0 += n
    ycat = jnp.concatenate(ys, axis=0)
    fused = jnp.dot(u_ref[...], ycat, preferred_element_type=jnp.float32,
                    precision=jax.lax.Precision.HIGHEST)
    out = jnp.dot(fused.astype(jnp.bfloat16), ow_ref[...],
                  preferred_element_type=jnp.float32) + ob_ref[...]
    o_ref[...] = jnp.maximum(out, 0.0).astype(o_ref.dtype)


def _sppm(x5, pool_w, pool_b, out_w, out_b, sizes):
    B, Ht, Wt, cin = x5.shape
    C = out_b.shape[-1]
    nb = len(sizes)
    npx = sum(ps * ps for ps in sizes)
    pix = [x5[:, ::Ht // ps, ::Wt // ps].reshape(B, ps * ps, cin)
           for ps in sizes]
    A = jnp.concatenate(pix, axis=1).astype(jnp.bfloat16)
    U = np.concatenate(
        [np.kron(_bilinear_matrix(Ht, ps), _bilinear_matrix(Wt, ps))
         for ps in sizes], axis=1)
    U = jnp.asarray(U, jnp.float32)
    out = pl.pallas_call(
        functools.partial(_sppm_body, cin=cin, nb=nb, sizes=sizes),
        out_shape=jax.ShapeDtypeStruct((B, Ht * Wt, C), jnp.bfloat16),
        grid=(B,),
        in_specs=[
            pl.BlockSpec((None, npx, cin), lambda b: (b, 0, 0)),
            pl.BlockSpec(pool_w.shape, lambda b: (0, 0)),
            pl.BlockSpec((1, C), lambda b: (0, 0)),
            pl.BlockSpec((Ht * Wt, npx), lambda b: (0, 0)),
            pl.BlockSpec((C, C), lambda b: (0, 0)),
            pl.BlockSpec((1, C), lambda b: (0, 0)),
        ],
        out_specs=pl.BlockSpec((None, Ht * Wt, C), lambda b: (b, 0, 0)),
        compiler_params=pltpu.CompilerParams(
            dimension_semantics=("parallel",), vmem_limit_bytes=_VMEM_LIMIT),
    )(A, pool_w, pool_b, U, out_w, out_b)
    return out.reshape(B, Ht, Wt, C)


def _fuse_body(x_ref, w_ref, b_ref, wa_ref, ba_ref, o_ref, *, H, W, C):
    W2 = W + 2
    P = (H + 2) * W2
    P_ext = x_ref.shape[0]
    offs = _offs(W2)

    a = jnp.concatenate([x_ref[off:off + P, :] for off in offs], axis=-1)
    y = jnp.dot(a, w_ref[...], preferred_element_type=jnp.float32) + b_ref[...]
    y = jnp.maximum(y, 0.0)
    x1 = y[:, :C]
    x2 = y[:, C:]

    att = jnp.concatenate(
        [jnp.mean(x1, axis=-1, keepdims=True),
         jnp.max(x1, axis=-1, keepdims=True),
         jnp.mean(x2, axis=-1, keepdims=True),
         jnp.max(x2, axis=-1, keepdims=True)], axis=-1)

    q = jax.lax.broadcasted_iota(jnp.int32, (P, 1), 0)
    oy = q // W2
    ox = q - oy * W2
    mask = jnp.logical_and(oy < H, ox < W)
    lead = W2 + 1
    trail = P_ext - P - lead
    att_full = jnp.concatenate(
        [jnp.zeros((lead, 4), jnp.float32),
         jnp.where(mask, att, 0.0),
         jnp.zeros((trail, 4), jnp.float32)], axis=0)
    acc = jnp.zeros((P, 1), jnp.float32)
    for s, off in enumerate(offs):
        acc = acc + jnp.sum(att_full[off:off + P, :] * wa_ref[s],
                            axis=-1, keepdims=True)
    alpha = jax.nn.sigmoid(acc + ba_ref[...])
    o_ref[...] = (x1 * alpha + x2 * (1.0 - alpha)).astype(o_ref.dtype)


def _fuse_level(feat, x_prev, w, bfull, wa, ba):
    B, H, W, cin = feat.shape
    C = bfull.shape[-1] // 2
    cinp = (w.shape[0] - 9 * C) // 9
    W2 = W + 2
    P = (H + 2) * W2
    p_ext = _ru(P + 2 * W2 + 2, 8)
    if cinp > cin:
        feat = jnp.pad(feat, ((0, 0), (0, 0), (0, 0), (0, cinp - cin)))
    x_up = jnp.repeat(jnp.repeat(x_prev, 2, axis=1), 2, axis=2)
    xcat = jnp.concatenate([feat.astype(jnp.bfloat16),
                            x_up.astype(jnp.bfloat16)], axis=-1)
    xp = _pad_flat(xcat, p_ext)
    K1 = 9 * cinp
    w_cat = jnp.concatenate(
        [w[:K1].reshape(9, cinp, 2 * C), w[K1:].reshape(9, C, 2 * C)],
        axis=1).reshape(9 * (cinp + C), 2 * C)
    out = pl.pallas_call(
        functools.partial(_fuse_body, H=H, W=W, C=C),
        out_shape=jax.ShapeDtypeStruct((B, P, C), jnp.bfloat16),
        grid=(B,),
        in_specs=[
            pl.BlockSpec((None, p_ext, cinp + C), lambda b: (b, 0, 0)),
            pl.BlockSpec((9 * (cinp + C), 2 * C), lambda b: (0, 0)),
            pl.BlockSpec((1, 2 * C), lambda b: (0, 0)),
            pl.BlockSpec((9, 1, 4), lambda b: (0, 0, 0)),
            pl.BlockSpec((1, 1), lambda b: (0, 0)),
        ],
        out_specs=pl.BlockSpec((None, P, C), lambda b: (b, 0, 0)),
        compiler_params=pltpu.CompilerParams(
            dimension_semantics=("parallel",), vmem_limit_bytes=_VMEM_LIMIT),
    )(xp, w_cat, bfull, wa, ba)
    return out.reshape(B, H + 2, W2, C)[:, :H, :W, :]


def _out_body(x_ref, w_ref, b_ref, wp_ref, bp_ref, s_ref, c_ref, *, H, W):
    W2 = W + 2
    P = (H + 2) * W2
    P_ext = x_ref.shape[0]
    C = x_ref.shape[-1]
    L = w_ref.shape[0]
    ncls = wp_ref.shape[-1]
    offs = _offs(W2)
    lead = W2 + 1
    trail = P_ext - P - lead

    q = jax.lax.broadcasted_iota(jnp.int32, (P, 1), 0)
    oy = q // W2
    ox = q - oy * W2
    mask = jnp.logical_and(oy < H, ox < W)

    def conv(src, wl, bl):
        a = jnp.concatenate([src[off:off + P, :] for off in offs], axis=-1)
        return jnp.maximum(
            jnp.dot(a, wl, preferred_element_type=jnp.float32) + bl, 0.0)

    y = conv(x_ref, w_ref[0], b_ref[0])
    for l in range(1, L):
        yb = jnp.where(mask, y, 0.0).astype(jnp.bfloat16)
        y_full = jnp.concatenate(
            [jnp.zeros((lead, C), jnp.bfloat16),
             yb,
             jnp.zeros((trail, C), jnp.bfloat16)], axis=0)
        y = conv(y_full, w_ref[l], b_ref[l])

    logits = jnp.dot(y.astype(jnp.bfloat16), wp_ref[...],
                     preferred_element_type=jnp.float32) + bp_ref[...]
    m = jnp.max(logits, axis=-1, keepdims=True)
    denom = jnp.sum(jnp.exp(logits - m), axis=-1, keepdims=True)
    s_ref[...] = (1.0 / denom).astype(s_ref.dtype)
    cidx = jax.lax.broadcasted_iota(jnp.int32, logits.shape, 1)
    c_ref[...] = jnp.min(jnp.where(logits == m, cidx, ncls),
                         axis=-1, keepdims=True).astype(c_ref.dtype)


def _out_chain(x, w_blocks, b_blocks, w_proj, b_proj):
    B, H, W, C = x.shape
    ncls = w_proj.shape[1]
    L = w_blocks.shape[0]
    W2 = W + 2
    P = (H + 2) * W2
    p_ext = _ru(P + 2 * W2 + 2, 8)
    x_pad = _pad_flat(x, p_ext)
    scores, classes = pl.pallas_call(
        functools.partial(_out_body, H=H, W=W),
        out_shape=(jax.ShapeDtypeStruct((B, P, 1), jnp.float32),
                   jax.ShapeDtypeStruct((B, P, 1), jnp.int32)),
        grid=(B,),
        in_specs=[
            pl.BlockSpec((None, p_ext, C), lambda b: (b, 0, 0)),
            pl.BlockSpec((L, 9 * C, C), lambda b: (0, 0, 0)),
            pl.BlockSpec((L, 1, C), lambda b: (0, 0, 0)),
            pl.BlockSpec((C, ncls), lambda b: (0, 0)),
            pl.BlockSpec((1, ncls), lambda b: (0, 0)),
        ],
        out_specs=(pl.BlockSpec((None, P, 1), lambda b: (b, 0, 0)),
                   pl.BlockSpec((None, P, 1), lambda b: (b, 0, 0))),
        compiler_params=pltpu.CompilerParams(
            dimension_semantics=("parallel",), vmem_limit_bytes=_VMEM_LIMIT),
    )(x_pad, w_blocks, b_blocks, w_proj, b_proj)
    scores = scores.reshape(B, H + 2, W2)[:, :H, :W]
    classes = classes.reshape(B, H + 2, W2)[:, :H, :W]
    return scores, classes


def _up_body(s_ref, c_ref, e_ref, et_ref, so_ref, co_ref):
    e = e_ref[...]
    et = et_ref[...]
    s = jnp.dot(e, s_ref[...], preferred_element_type=jnp.float32,
                precision=jax.lax.Precision.HIGHEST)
    so_ref[...] = jnp.dot(s, et, preferred_element_type=jnp.float32,
                          precision=jax.lax.Precision.HIGHEST)
    c = jnp.dot(e, c_ref[...], preferred_element_type=jnp.float32)
    co_ref[...] = jnp.dot(c, et, preferred_element_type=jnp.float32).astype(
        jnp.int32)


def _upsample(scores, classes, r):
    B, H, W = scores.shape
    E = np.zeros((H * r, H), np.float32)
    E[np.arange(H * r), np.arange(H * r) // r] = 1.0
    Ej = jnp.asarray(E)
    return pl.pallas_call(
        _up_body,
        out_shape=(jax.ShapeDtypeStruct((B, H * r, W * r), jnp.float32),
                   jax.ShapeDtypeStruct((B, H * r, W * r), jnp.int32)),
        grid=(B,),
        in_specs=[
            pl.BlockSpec((None, H, W), lambda b: (b, 0, 0)),
            pl.BlockSpec((None, H, W), lambda b: (b, 0, 0)),
            pl.BlockSpec((H * r, H), lambda b: (0, 0)),
            pl.BlockSpec((W, W * r), lambda b: (0, 0)),
        ],
        out_specs=(pl.BlockSpec((None, H * r, W * r), lambda b: (b, 0, 0)),
                   pl.BlockSpec((None, H * r, W * r), lambda b: (b, 0, 0))),
        compiler_params=pltpu.CompilerParams(
            dimension_semantics=("parallel",), vmem_limit_bytes=_VMEM_LIMIT),
    )(scores, classes.astype(jnp.float32), Ej, Ej.T)


def kernel(sppm_pool_w, sppm_pool_b, sppm_out_w, sppm_out_b,
           out_w_blocks, out_b_blocks, out_w_proj, out_b_proj,
           level0_w, level0_b, level0_w_att, level0_b_att,
           level1_w, level1_b, level1_w_att, level1_b_att,
           input_0, input_1, input_2, input_3, input_4, input_5):
    x5 = jnp.transpose(input_5, (0, 2, 3, 1))
    x4 = jnp.transpose(input_4, (0, 2, 3, 1))
    x3 = jnp.transpose(input_3, (0, 2, 3, 1))

    x = _sppm(x5, sppm_pool_w, sppm_pool_b, sppm_out_w, sppm_out_b, (1, 2, 4))
    x = _fuse_level(x4, x, level0_w, level0_b, level0_w_att, level0_b_att)
    x = _fuse_level(x3, x, level1_w, level1_b, level1_w_att, level1_b_att)
    scores, classes = _out_chain(x, out_w_blocks, out_b_blocks,
                                 out_w_proj, out_b_proj)

    H0, W0 = input_0.shape[2], input_0.shape[3]
    return _upsample(scores, classes, H0 // scores.shape[1])


# speedup vs baseline: 4.8874x; 4.1445x over previous
import functools

import jax
import jax.numpy as jnp
import numpy as np
from jax.experimental import pallas as pl
from jax.experimental.pallas import tpu as pltpu

_VMEM_LIMIT = 100 * 1024 * 1024
_HIGHEST = jax.lax.Precision.HIGHEST


def _ru(x, m):
    return ((x + m - 1) // m) * m


def _bilinear_matrix(out, inn):
    c = (np.arange(out, dtype=np.float64) + 0.5) * (inn / out) - 0.5
    c = np.clip(c, 0.0, inn - 1)
    lo = np.floor(c).astype(np.int32)
    hi = np.minimum(lo + 1, inn - 1)
    f = (c - lo).astype(np.float32)
    R = np.zeros((out, inn), np.float32)
    R[np.arange(out), lo] += 1.0 - f
    R[np.arange(out), hi] += f
    return R


def _up_pad_matrix(rows, stride, H, W, Q):
    R = np.zeros((rows, Q), np.float32)
    W2 = W + 2
    for oy in range(H):
        for ox in range(W):
            R[(oy // 2) * stride + (ox // 2), (oy + 1) * W2 + (ox + 1)] = 1.0
    return R


def _conv_t(src, wT, bT, s_offs, P):
    a = jnp.concatenate([src[:, off:off + P] for off in s_offs], axis=0)
    y = jnp.dot(wT, a, preferred_element_type=jnp.float32) + bT
    return jnp.maximum(y, 0.0)


def _embed_t(x, mask, lead, Q, P, dtype):
    C = x.shape[0]
    xm = (x * mask).astype(dtype)
    return jnp.concatenate(
        [jnp.zeros((C, lead), dtype), xm,
         jnp.zeros((C, Q - P - lead), dtype)], axis=1)


def _att_alpha(x1, x2, wa_ref, ba_ref, mask, s_offs, lead, Q, P):
    att = jnp.concatenate(
        [jnp.mean(x1, axis=0, keepdims=True),
         jnp.max(x1, axis=0, keepdims=True),
         jnp.mean(x2, axis=0, keepdims=True),
         jnp.max(x2, axis=0, keepdims=True)], axis=0)
    att_full = _embed_t(att, mask, lead, Q, P, jnp.float32)
    acc = jnp.zeros((4, P), jnp.float32)
    for s, off in enumerate(s_offs):
        acc = acc + att_full[:, off:off + P] * wa_ref[s]
    a1 = jnp.sum(acc, axis=0, keepdims=True) + ba_ref[...]
    return jax.nn.sigmoid(a1)


def _seg_body(x5_ref, x4_ref, x3_ref,
              pwT_ref, pbT_ref, sel_ref, uT_ref, owT_ref, obT_ref,
              r4_ref, r3_ref,
              w4lat_ref, w4up_ref, b4T_ref, wa4_ref, ba4_ref,
              w3lat_ref, w3up_ref, b3T_ref, wa3_ref, ba3_ref,
              wcT_ref, bcT_ref, wpT_ref, bpT_ref,
              o_ref,
              *, sizes, cin5, cin4, cin3, C, ncls):
    aT = jnp.dot(x5_ref[...], sel_ref[...],
                 preferred_element_type=jnp.float32)
    ys = []
    r0 = 0
    for bi, ps in enumerate(sizes):
        n = ps * ps
        wbT = pwT_ref[:, bi * cin5:(bi + 1) * cin5]
        bbT = pwT_ref[:, 3 * cin5 + bi:3 * cin5 + bi + 1].astype(jnp.float32)
        y = jnp.dot(wbT, aT[:, r0:r0 + n].astype(jnp.bfloat16),
                    preferred_element_type=jnp.float32)
        ys.append(jnp.maximum(y + bbT + pbT_ref[...], 0.0))
        r0 += n
    ycatT = jnp.concatenate(ys, axis=1)
    fusedT = jnp.dot(ycatT, uT_ref[...],
                     preferred_element_type=jnp.float32, precision=_HIGHEST)
    xT = jnp.dot(owT_ref[...], fusedT.astype(jnp.bfloat16),
                 preferred_element_type=jnp.float32) + obT_ref[...]
    xT = jnp.maximum(xT, 0.0)

    H4, W4 = 32, 32
    W24 = W4 + 2
    P4 = (H4 + 2) * W24
    offs4 = [dy * W24 + dx for dy in range(3) for dx in range(3)]
    lead4 = W24 + 1
    Q4 = r4_ref.shape[1]
    up4 = jnp.dot(xT.astype(jnp.bfloat16), r4_ref[...],
                  preferred_element_type=jnp.float32).astype(jnp.bfloat16)
    q4 = jax.lax.broadcasted_iota(jnp.int32, (1, P4), 1)
    oy4 = q4 // W24
    ox4 = q4 - oy4 * W24
    mask4 = jnp.logical_and(oy4 < H4, ox4 < W4).astype(jnp.float32)
    x1 = _conv_t(x4_ref[...], w4lat_ref[...], b4T_ref[:, :1], offs4, P4)
    x2 = _conv_t(up4, w4up_ref[...], b4T_ref[:, 1:], offs4, P4)
    alpha = _att_alpha(x1, x2, wa4_ref, ba4_ref, mask4, offs4, lead4, Q4, P4)
    o4 = x1 * alpha + x2 * (1.0 - alpha)

    H3, W3 = 64, 64
    W23 = W3 + 2
    P3 = (H3 + 2) * W23
    offs3 = [dy * W23 + dx for dy in range(3) for dx in range(3)]
    lead3 = W23 + 1
    Q3 = r3_ref.shape[1]
    up3 = jnp.dot(o4.astype(jnp.bfloat16), r3_ref[...],
                  preferred_element_type=jnp.float32).astype(jnp.bfloat16)
    q3 = jax.lax.broadcasted_iota(jnp.int32, (1, P3), 1)
    oy3 = q3 // W23
    ox3 = q3 - oy3 * W23
    mask3 = jnp.logical_and(oy3 < H3, ox3 < W3).astype(jnp.float32)
    x1 = _conv_t(x3_ref[...], w3lat_ref[...], b3T_ref[:, :1], offs3, P3)
    x2 = _conv_t(up3, w3up_ref[...], b3T_ref[:, 1:], offs3, P3)
    alpha = _att_alpha(x1, x2, wa3_ref, ba3_ref, mask3, offs3, lead3, Q3, P3)
    o3 = x1 * alpha + x2 * (1.0 - alpha)

    L = wcT_ref.shape[0]
    y = o3
    for l in range(L):
        y_full = _embed_t(y, mask3, lead3, Q3, P3, jnp.bfloat16)
        y = _conv_t(y_full, wcT_ref[l], bcT_ref[l], offs3, P3)
    logits = jnp.dot(wpT_ref[...], y.astype(jnp.bfloat16),
                     preferred_element_type=jnp.float32) + bpT_ref[...]
    m = jnp.max(logits, axis=0, keepdims=True)
    denom = jnp.sum(jnp.exp(logits - m), axis=0, keepdims=True)
    score = 1.0 / denom
    cidx = jax.lax.broadcasted_iota(jnp.int32, logits.shape, 0).astype(
        jnp.float32)
    cls = jnp.min(jnp.where(logits == m, cidx, float(ncls)),
                  axis=0, keepdims=True)
    pad = jnp.zeros((o_ref.shape[0] - 2, P3), jnp.float32)
    o_ref[...] = jnp.concatenate([score, cls, pad], axis=0)


def _seg_head(x5r, x4p, x3p, consts):
    B = x5r.shape[0]
    P3 = 66 * 66
    specs = [pl.BlockSpec((None,) + x.shape[1:], lambda b: (b, 0, 0))
             for x in (x5r, x4p, x3p)]
    specs += [pl.BlockSpec(c.shape, lambda b, n=c.ndim: (0,) * n)
              for c in consts]
    out = pl.pallas_call(
        functools.partial(_seg_body, sizes=(1, 2, 4), cin5=32, cin4=24,
                          cin3=16, C=32, ncls=5),
        out_shape=jax.ShapeDtypeStruct((B, 8, P3), jnp.float32),
        grid=(B,),
        in_specs=specs,
        out_specs=pl.BlockSpec((None, 8, P3), lambda b: (b, 0, 0)),
        compiler_params=pltpu.CompilerParams(
            dimension_semantics=("parallel",), vmem_limit_bytes=_VMEM_LIMIT),
    )(x5r, x4p, x3p, *consts)
    return out


def _up_body(s_ref, c_ref, e_ref, et_ref, so_ref, co_ref):
    e = e_ref[...]
    et = et_ref[...]
    s = jnp.dot(e, s_ref[...], preferred_element_type=jnp.float32,
                precision=_HIGHEST)
    so_ref[...] = jnp.dot(s, et, preferred_element_type=jnp.float32,
                          precision=_HIGHEST)
    c = jnp.dot(e, c_ref[...], preferred_element_type=jnp.float32)
    co_ref[...] = jnp.dot(c, et, preferred_element_type=jnp.float32).astype(
        jnp.int32)


def _upsample(scores, classes, r):
    B, H, W = scores.shape
    E = np.zeros((H * r, H), np.float32)
    E[np.arange(H * r), np.arange(H * r) // r] = 1.0
    Ej = jnp.asarray(E)
    return pl.pallas_call(
        _up_body,
        out_shape=(jax.ShapeDtypeStruct((B, H * r, W * r), jnp.float32),
                   jax.ShapeDtypeStruct((B, H * r, W * r), jnp.int32)),
        grid=(B,),
        in_specs=[
            pl.BlockSpec((None, H, W), lambda b: (b, 0, 0)),
            pl.BlockSpec((None, H, W), lambda b: (b, 0, 0)),
            pl.BlockSpec((H * r, H), lambda b: (0, 0)),
            pl.BlockSpec((W, W * r), lambda b: (0, 0)),
        ],
        out_specs=(pl.BlockSpec((None, H * r, W * r), lambda b: (b, 0, 0)),
                   pl.BlockSpec((None, H * r, W * r), lambda b: (b, 0, 0))),
        compiler_params=pltpu.CompilerParams(
            dimension_semantics=("parallel",), vmem_limit_bytes=_VMEM_LIMIT),
    )(scores, classes, Ej, Ej.T)


def _pad_flat_t(x, q):
    B, C, H, W = x.shape
    xp = jnp.pad(x, ((0, 0), (0, 0), (1, 1), (1, 1)))
    flat = xp.reshape(B, C, (H + 2) * (W + 2)).astype(jnp.bfloat16)
    return jnp.pad(flat, ((0, 0), (0, 0), (0, q - flat.shape[-1])))


def kernel(sppm_pool_w, sppm_pool_b, sppm_out_w, sppm_out_b,
           out_w_blocks, out_b_blocks, out_w_proj, out_b_proj,
           level0_w, level0_b, level0_w_att, level0_b_att,
           level1_w, level1_b, level1_w_att, level1_b_att,
           input_0, input_1, input_2, input_3, input_4, input_5):
    B = input_5.shape[0]
    C = 32
    sizes = (1, 2, 4)
    Q4 = _ru(1156 + 2 * 34 + 2, 128)
    Q3 = _ru(4356 + 2 * 66 + 2, 128)

    x5r = input_5.reshape(B, C, 256).astype(jnp.bfloat16)
    x4p = _pad_flat_t(input_4, Q4)
    x3p = _pad_flat_t(input_3, Q3)

    sel = np.zeros((256, 21), np.float32)
    col = 0
    for ps in sizes:
        st = 16 // ps
        for yy in range(ps):
            for xx in range(ps):
                sel[(yy * st) * 16 + xx * st, col] = 1.0
                col += 1
    U = np.concatenate(
        [np.kron(_bilinear_matrix(16, ps), _bilinear_matrix(16, ps))
         for ps in sizes], axis=1)
    uT = jnp.asarray(U.T, jnp.float32)
    selj = jnp.asarray(sel, jnp.bfloat16)
    pwT = sppm_pool_w.T
    pbT = sppm_pool_b.T
    owT = sppm_out_w.T
    obT = sppm_out_b.T

    r4 = jnp.asarray(_up_pad_matrix(256, 16, 32, 32, Q4), jnp.bfloat16)
    r3 = jnp.asarray(_up_pad_matrix(1156, 34, 64, 64, Q3), jnp.bfloat16)

    def _level_w(w, bfull, cin):
        cinp = (w.shape[0] - 9 * C) // 9
        wlat = w[:9 * cinp].reshape(9, cinp, 2 * C)[:, :cin, :C]
        wlatT = jnp.transpose(wlat, (2, 0, 1)).reshape(C, 9 * cin)
        wup = w[9 * cinp:].reshape(9, C, 2 * C)[:, :, C:]
        wupT = jnp.transpose(wup, (2, 0, 1)).reshape(C, 9 * C)
        bT = jnp.stack([bfull[0, :C], bfull[0, C:]], axis=1)
        return wlatT, wupT, bT

    w4lat, w4up, b4T = _level_w(level0_w, level0_b, 24)
    w3lat, w3up, b3T = _level_w(level1_w, level1_b, 16)
    wa4 = jnp.transpose(level0_w_att, (0, 2, 1))
    wa3 = jnp.transpose(level1_w_att, (0, 2, 1))

    wcT = jnp.transpose(out_w_blocks, (0, 2, 1))
    bcT = jnp.transpose(out_b_blocks, (0, 2, 1))
    wpT = out_w_proj.T
    bpT = out_b_proj.T

    consts = (pwT, pbT, selj, uT, owT, obT, r4, r3,
              w4lat, w4up, b4T, wa4, level0_b_att,
              w3lat, w3up, b3T, wa3, level1_b_att,
              wcT, bcT, wpT, bpT)
    out = _seg_head(x5r, x4p, x3p, consts)

    sc = out[:, 0].reshape(B, 66, 66)[:, :64, :64]
    cl = out[:, 1].reshape(B, 66, 66)[:, :64, :64]
    H0 = input_0.shape[2]
    return _upsample(sc, cl, H0 // 64)
